# Initial kernel scaffold; baseline (speedup 1.0000x reference)
import jax
import jax.numpy as jnp
from jax.experimental import pallas as pl


def kernel(x, edge_d, edge_index, W, b):
    raise NotImplementedError("write your pallas kernel here")


# speedup vs baseline: 6.2578x; 6.2578x over previous
import functools

import jax
import jax.numpy as jnp
from jax import lax
from jax.experimental import pallas as pl
from jax.experimental.pallas import tpu as pltpu
from jax.experimental.pallas import tpu_sc as plsc

N_NODES = 10000
N_EDGES = 320000
D = 128

NC = 2
NS = 16
NW = NC * NS
EPW = N_EDGES // NW
EB = 80
NBATCH = EPW // EB
ROWS_PAD = 10240
RPT = ROWS_PAD // NS


def _sc_body(src_hbm, dst_hbm, d_hbm, x_hbm, s_out, deg_out,
             acc, degs, rows_v, src_v, dst_v, d_v, ones_v, zbuf, gsem):
    c = lax.axis_index("c")
    s = lax.axis_index("s")
    wid = c * NS + s

    def zloop(i, _):
        zero16 = jnp.zeros((16,), jnp.float32)
        for cc in range(8):
            zbuf[i, pl.ds(cc * 16, 16)] = zero16
        return 0
    lax.fori_loop(0, 128, zloop, 0)
    for k in range(RPT // 128):
        pltpu.sync_copy(zbuf, acc.at[pl.ds(s * RPT + k * 128, 128)])
    for k in range(RPT // 128):
        pltpu.sync_copy(zbuf.at[0], degs.at[pl.ds(s * RPT + k * 128, 128)])

    def oloop(i, _):
        ones_v[pl.ds(i * 16, 16)] = jnp.ones((16,), jnp.float32)
        return 0
    lax.fori_loop(0, EB // 16, oloop, 0)

    plsc.subcore_barrier()

    ebase = wid * EPW

    def batch(t, _):
        eb = ebase + t * EB
        pltpu.sync_copy(src_hbm.at[pl.ds(eb, EB)], src_v)
        pltpu.sync_copy(dst_hbm.at[pl.ds(eb, EB)], dst_v)
        pltpu.sync_copy(d_hbm.at[pl.ds(eb, EB)], d_v)
        pltpu.async_copy(x_hbm.at[src_v], rows_v, gsem).wait()

        def scale(g, _):
            svec = 1.0 - d_v[pl.ds(g * 16, 16)]
            for j in range(16):
                sc = svec[j]
                row = g * 16 + j
                for cc in range(8):
                    rows_v[row, pl.ds(cc * 16, 16)] = (
                        rows_v[row, pl.ds(cc * 16, 16)] * sc)
            return 0
        lax.fori_loop(0, EB // 16, scale, 0)

        pltpu.sync_copy(rows_v, acc.at[dst_v], add=True)
        pltpu.sync_copy(ones_v, degs.at[dst_v], add=True)
        return 0
    lax.fori_loop(0, NBATCH, batch, 0)

    plsc.subcore_barrier()

    pltpu.sync_copy(acc.at[pl.ds(s * RPT, RPT)], s_out.at[c].at[pl.ds(s * RPT, RPT)])
    pltpu.sync_copy(degs.at[pl.ds(s * RPT, RPT)], deg_out.at[c].at[pl.ds(s * RPT, RPT)])


def _sc_scatter(x, edge_d, src, dst):
    mesh = plsc.VectorSubcoreMesh(core_axis_name="c", subcore_axis_name="s")
    f = pl.kernel(
        _sc_body,
        out_type=(
            jax.ShapeDtypeStruct((NC, ROWS_PAD, D), jnp.float32),
            jax.ShapeDtypeStruct((NC, ROWS_PAD), jnp.float32),
        ),
        mesh=mesh,
        scratch_types=[
            pltpu.VMEM_SHARED((ROWS_PAD, D), jnp.float32),
            pltpu.VMEM_SHARED((ROWS_PAD,), jnp.float32),
            pltpu.VMEM((EB, D), jnp.float32),
            pltpu.VMEM((EB,), jnp.int32),
            pltpu.VMEM((EB,), jnp.int32),
            pltpu.VMEM((EB,), jnp.float32),
            pltpu.VMEM((EB,), jnp.float32),
            pltpu.VMEM((128, D), jnp.float32),
            pltpu.SemaphoreType.DMA,
        ],
    )
    return f(src, dst, edge_d, x)


def _tc_body(x_ref, s0_ref, s1_ref, g0_ref, g1_ref, w_ref, b_ref, o_ref):
    deg = jnp.maximum(g0_ref[...] + g1_ref[...], 1.0)
    h = s0_ref[...] + s1_ref[...] + deg * x_ref[...]
    y = jnp.dot(h, w_ref[...], preferred_element_type=jnp.float32)
    o_ref[...] = jnp.maximum(y + b_ref[...], 0.0)


def _tc_final(x, s0, s1, g0, g1, W, b):
    BR = 400
    grid = (N_NODES // BR,)
    return pl.pallas_call(
        _tc_body,
        grid=grid,
        in_specs=[
            pl.BlockSpec((BR, D), lambda i: (i, 0)),
            pl.BlockSpec((BR, D), lambda i: (i, 0)),
            pl.BlockSpec((BR, D), lambda i: (i, 0)),
            pl.BlockSpec((BR, 1), lambda i: (i, 0)),
            pl.BlockSpec((BR, 1), lambda i: (i, 0)),
            pl.BlockSpec((D, D), lambda i: (0, 0)),
            pl.BlockSpec((1, D), lambda i: (0, 0)),
        ],
        out_specs=pl.BlockSpec((BR, D), lambda i: (i, 0)),
        out_shape=jax.ShapeDtypeStruct((N_NODES, D), jnp.float32),
    )(x, s0, s1, g0, g1, W, b)


@jax.jit
def kernel(x, edge_d, edge_index, W, b):
    src = edge_index[0]
    dst = edge_index[1]
    s_part, deg_part = _sc_scatter(x, edge_d, src, dst)
    s0 = s_part[0, :N_NODES]
    s1 = s_part[1, :N_NODES]
    g0 = deg_part[0, :N_NODES].reshape(N_NODES, 1)
    g1 = deg_part[1, :N_NODES].reshape(N_NODES, 1)
    return _tc_final(x, s0, s1, g0, g1, W, b.reshape(1, D))


# speedup vs baseline: 13.2008x; 2.1095x over previous
import jax
import jax.numpy as jnp
from jax import lax
from jax.experimental import pallas as pl
from jax.experimental.pallas import tpu as pltpu
from jax.experimental.pallas import tpu_sc as plsc

N_NODES = 10000
N_EDGES = 320000
D = 128

NC = 2
NS = 16
NW = NC * NS
EPW = N_EDGES // NW
EB = 80
NBATCH = EPW // EB
CHUNK = 25
NCHUNK = NBATCH // CHUNK
CE = CHUNK * EB
ROWS_PAD = 10240
RPT = ROWS_PAD // NS


def _sc_body(src4, dst4, d4, x_hbm, s_out, deg_out,
             acc, degs, rows0, rows1, srcs, dsts, ds_all, ones_v,
             gsem0, gsem1, ssem0, ssem1, dsem):
    c = lax.axis_index("c")
    s = lax.axis_index("s")
    wid = c * NS + s

    def zloop(i, _):
        zero16 = jnp.zeros((16,), jnp.float32)
        for cc in range(8):
            rows0[i, pl.ds(cc * 16, 16)] = zero16
        return 0
    lax.fori_loop(0, EB, zloop, 0)
    for k in range(RPT // EB):
        pltpu.sync_copy(rows0, acc.at[pl.ds(s * RPT + k * EB, EB)])
    for k in range(RPT // D):
        pltpu.sync_copy(rows0.at[0], degs.at[pl.ds(s * RPT + k * D, D)])

    def oloop(i, _):
        ones_v[pl.ds(i * 16, 16)] = jnp.ones((16,), jnp.float32)
        return 0
    lax.fori_loop(0, EB // 16, oloop, 0)

    plsc.subcore_barrier()

    def scale(rows, t):
        def body(g, _):
            svec = 1.0 - ds_all[t, pl.ds(g * 16, 16)]
            for j in range(16):
                sc = svec[j]
                row = g * 16 + j
                for cc in range(8):
                    rows[row, pl.ds(cc * 16, 16)] = (
                        rows[row, pl.ds(cc * 16, 16)] * sc)
            return 0
        lax.fori_loop(0, EB // 16, body, 0)

    def start_gather(t, rows, gsem):
        pltpu.async_copy(x_hbm.at[srcs.at[t]], rows, gsem)

    def start_scatter(t, rows, ssem):
        pltpu.async_copy(rows, acc.at[dsts.at[t]], ssem, add=True)
        pltpu.async_copy(ones_v, degs.at[dsts.at[t]], dsem, add=True)

    def wait_gather(rows, gsem):
        pltpu.make_async_copy(x_hbm.at[srcs.at[0]], rows, gsem).wait()

    def wait_scatter(rows, ssem):
        pltpu.make_async_copy(rows, acc.at[dsts.at[0]], ssem).wait()

    def do_batch(t, rows, gsem, ssem, tg):
        wait_gather(rows, gsem)
        scale(rows, t)
        start_scatter(t, rows, ssem)
        wait_scatter(rows, ssem)
        if tg is not None:
            start_gather(tg, rows, gsem)

    for k in range(NCHUNK):
        pltpu.sync_copy(src4.at[wid].at[k], srcs)
        pltpu.sync_copy(dst4.at[wid].at[k], dsts)
        pltpu.sync_copy(d4.at[wid].at[k], ds_all)

        start_gather(0, rows0, gsem0)
        start_gather(1, rows1, gsem1)

        def pipe(t2, _):
            ta = 2 * t2
            do_batch(ta, rows0, gsem0, ssem0, ta + 2)
            do_batch(ta + 1, rows1, gsem1, ssem1, ta + 3)
            return 0
        lax.fori_loop(0, (CHUNK - 3) // 2, pipe, 0)

        do_batch(CHUNK - 3, rows0, gsem0, ssem0, CHUNK - 1)
        do_batch(CHUNK - 2, rows1, gsem1, ssem1, None)
        do_batch(CHUNK - 1, rows0, gsem0, ssem0, None)

        def ddrain(t, _):
            pltpu.make_async_copy(ones_v, degs.at[dsts.at[0]], dsem).wait()
            return 0
        lax.fori_loop(0, CHUNK, ddrain, 0)

    plsc.subcore_barrier()

    pltpu.sync_copy(acc.at[pl.ds(s * RPT, RPT)],
                    s_out.at[c].at[pl.ds(s * RPT, RPT)])
    pltpu.sync_copy(degs.at[pl.ds(s * RPT, RPT)],
                    deg_out.at[c].at[pl.ds(s * RPT, RPT)])


def _sc_scatter(x, edge_d, src, dst):
    mesh = plsc.VectorSubcoreMesh(core_axis_name="c", subcore_axis_name="s")
    f = pl.kernel(
        _sc_body,
        out_type=(
            jax.ShapeDtypeStruct((NC, ROWS_PAD, D), jnp.float32),
            jax.ShapeDtypeStruct((NC, ROWS_PAD), jnp.float32),
        ),
        mesh=mesh,
        scratch_types=[
            pltpu.VMEM_SHARED((ROWS_PAD, D), jnp.float32),
            pltpu.VMEM_SHARED((ROWS_PAD,), jnp.float32),
            pltpu.VMEM((EB, D), jnp.float32),
            pltpu.VMEM((EB, D), jnp.float32),
            pltpu.VMEM((CHUNK, EB), jnp.int32),
            pltpu.VMEM((CHUNK, EB), jnp.int32),
            pltpu.VMEM((CHUNK, EB), jnp.float32),
            pltpu.VMEM((EB,), jnp.float32),
            pltpu.SemaphoreType.DMA,
            pltpu.SemaphoreType.DMA,
            pltpu.SemaphoreType.DMA,
            pltpu.SemaphoreType.DMA,
            pltpu.SemaphoreType.DMA,
        ],
    )
    src4 = src.reshape(NW, NCHUNK, CHUNK, EB)
    dst4 = dst.reshape(NW, NCHUNK, CHUNK, EB)
    d4 = edge_d.reshape(NW, NCHUNK, CHUNK, EB)
    return f(src4, dst4, d4, x)


def _tc_body(x_ref, s0_ref, s1_ref, g0_ref, g1_ref, w_ref, b_ref, o_ref):
    deg = jnp.maximum(g0_ref[...] + g1_ref[...], 1.0)
    h = s0_ref[...] + s1_ref[...] + deg * x_ref[...]
    y = jnp.dot(h, w_ref[...], preferred_element_type=jnp.float32)
    o_ref[...] = jnp.maximum(y + b_ref[...], 0.0)


def _tc_final(x, s0, s1, g0, g1, W, b):
    BR = 400
    grid = (N_NODES // BR,)
    return pl.pallas_call(
        _tc_body,
        grid=grid,
        in_specs=[
            pl.BlockSpec((BR, D), lambda i: (i, 0)),
            pl.BlockSpec((BR, D), lambda i: (i, 0)),
            pl.BlockSpec((BR, D), lambda i: (i, 0)),
            pl.BlockSpec((BR, 1), lambda i: (i, 0)),
            pl.BlockSpec((BR, 1), lambda i: (i, 0)),
            pl.BlockSpec((D, D), lambda i: (0, 0)),
            pl.BlockSpec((1, D), lambda i: (0, 0)),
        ],
        out_specs=pl.BlockSpec((BR, D), lambda i: (i, 0)),
        out_shape=jax.ShapeDtypeStruct((N_NODES, D), jnp.float32),
    )(x, s0, s1, g0, g1, W, b)


@jax.jit
def kernel(x, edge_d, edge_index, W, b):
    src = edge_index[0]
    dst = edge_index[1]
    s_part, deg_part = _sc_scatter(x, edge_d, src, dst)
    s0 = s_part[0, :N_NODES]
    s1 = s_part[1, :N_NODES]
    g0 = deg_part[0, :N_NODES].reshape(N_NODES, 1)
    g1 = deg_part[1, :N_NODES].reshape(N_NODES, 1)
    return _tc_final(x, s0, s1, g0, g1, W, b.reshape(1, D))


# speedup vs baseline: 15.9613x; 1.2091x over previous
import jax
import jax.numpy as jnp
from jax import lax
from jax.experimental import pallas as pl
from jax.experimental.pallas import tpu as pltpu
from jax.experimental.pallas import tpu_sc as plsc

N_NODES = 10000
N_EDGES = 320000
D = 128

NC = 2
NS = 16
NW = NC * NS
EPW = N_EDGES // NW
EB = 80
NBATCH = EPW // EB
CHUNK = 25
NCHUNK = NBATCH // CHUNK
CE = CHUNK * EB
NBUF = 3
PF = 2
ROWS_PAD = 10240
RPT = ROWS_PAD // NS
DEG_PAD = ROWS_PAD
RPT_D = RPT


def _sc_body(ei, ed, x_hbm, s_out, deg_out,
             acc, degs, r0, r1, r2, srcs, dsts, dsv, ones_v,
             g0, g1, g2, s0, s1, s2, dsem):
    c = lax.axis_index("c")
    s = lax.axis_index("s")
    wid = c * NS + s
    rows = [r0, r1, r2]
    gsem = [g0, g1, g2]
    ssem = [s0, s1, s2]

    def zloop(i, _):
        zero16 = jnp.zeros((16,), jnp.float32)
        for cc in range(8):
            r0[i, pl.ds(cc * 16, 16)] = zero16
        return 0
    lax.fori_loop(0, EB, zloop, 0)
    for k in range(RPT // EB):
        pltpu.sync_copy(r0, acc.at[pl.ds(s * RPT + k * EB, EB)])
    for k in range(RPT_D // D):
        pltpu.sync_copy(r0.at[0], degs.at[pl.ds(s * RPT_D + k * D, D)])

    def oloop(i, _):
        ones_v[pl.ds(i * 16, 16)] = jnp.ones((16,), jnp.float32)
        return 0
    lax.fori_loop(0, EB // 16, oloop, 0)

    plsc.subcore_barrier()

    def start_gather(t, b):
        pltpu.async_copy(x_hbm.at[srcs.at[t]], rows[b], gsem[b])

    def wait_gather(b):
        pltpu.make_async_copy(x_hbm.at[srcs.at[0]], rows[b],
                              gsem[b]).wait()

    def start_scatter(t, b):
        pltpu.async_copy(rows[b], acc.at[dsts.at[t]], ssem[b], add=True)
        pltpu.async_copy(ones_v, degs.at[dsts.at[t]], dsem, add=True)

    def wait_scatter(b):
        pltpu.make_async_copy(rows[b], acc.at[dsts.at[0]],
                              ssem[b]).wait()

    def scale(t, b):
        r = rows[b]

        def body(g, _):
            svec = 1.0 - dsv[t, pl.ds(g * 16, 16)]
            for j in range(16):
                sc = svec[j]
                row = g * 16 + j
                for cc in range(8):
                    r[row, pl.ds(cc * 16, 16)] = (
                        r[row, pl.ds(cc * 16, 16)] * sc)
            return 0
        lax.fori_loop(0, EB // 16, body, 0)

    def step(t, b, ws, nxt):
        wait_gather(b)
        scale(t, b)
        start_scatter(t, b)
        b2 = (b + PF) % NBUF
        if ws:
            wait_scatter(b2)
        if nxt is not None:
            start_gather(nxt, b2)

    def chunk_body(k, _):
        pltpu.sync_copy(ei.at[0].at[wid].at[k], srcs)
        pltpu.sync_copy(ei.at[1].at[wid].at[k], dsts)
        pltpu.sync_copy(ed.at[wid].at[k], dsv)

        for t in range(PF):
            start_gather(t, t)
        step(0, 0, False, 2)
        step(1, 1, True, 3)
        step(2, 2, True, 4)

        def pipe(t3, _):
            ta = 3 * t3
            step(ta, 0, True, ta + PF)
            step(ta + 1, 1, True, ta + 1 + PF)
            step(ta + 2, 2, True, ta + 2 + PF)
            return 0
        lax.fori_loop(1, (CHUNK - 4) // NBUF, pipe, 0)

        step(21, 0, True, 23)
        step(22, 1, True, 24)
        step(23, 2, False, None)
        step(24, 0, False, None)
        wait_scatter(1)
        wait_scatter(2)
        wait_scatter(0)

        def ddrain(i, _):
            pltpu.make_async_copy(ones_v, degs.at[dsts.at[0]],
                                  dsem).wait()
            return 0
        lax.fori_loop(0, CHUNK, ddrain, 0)
        return 0

    lax.fori_loop(0, NCHUNK, chunk_body, 0)

    plsc.subcore_barrier()

    pltpu.sync_copy(acc.at[pl.ds(s * RPT, RPT)],
                    s_out.at[c].at[pl.ds(s * RPT, RPT)])
    pltpu.sync_copy(degs.at[pl.ds(s * RPT_D, RPT_D)],
                    deg_out.at[c].at[pl.ds(s * RPT_D, RPT_D)])


def _sc_scatter(x, edge_d, edge_index):
    mesh = plsc.VectorSubcoreMesh(core_axis_name="c", subcore_axis_name="s")
    f = pl.kernel(
        _sc_body,
        out_type=(
            jax.ShapeDtypeStruct((NC, ROWS_PAD, D), jnp.float32),
            jax.ShapeDtypeStruct((NC, DEG_PAD), jnp.float32),
        ),
        mesh=mesh,
        scratch_types=[
            pltpu.VMEM_SHARED((ROWS_PAD, D), jnp.float32),
            pltpu.VMEM_SHARED((DEG_PAD,), jnp.float32),
            pltpu.VMEM((EB, D), jnp.float32),
            pltpu.VMEM((EB, D), jnp.float32),
            pltpu.VMEM((EB, D), jnp.float32),
            pltpu.VMEM((CHUNK, EB), jnp.int32),
            pltpu.VMEM((CHUNK, EB), jnp.int32),
            pltpu.VMEM((CHUNK, EB), jnp.float32),
            pltpu.VMEM((EB,), jnp.float32),
            pltpu.SemaphoreType.DMA,
            pltpu.SemaphoreType.DMA,
            pltpu.SemaphoreType.DMA,
            pltpu.SemaphoreType.DMA,
            pltpu.SemaphoreType.DMA,
            pltpu.SemaphoreType.DMA,
            pltpu.SemaphoreType.DMA,
        ],
    )
    return f(edge_index.reshape(2, NW, NCHUNK, CHUNK, EB),
             edge_d.reshape(NW, NCHUNK, CHUNK, EB), x)


def _tc_body(x_ref, s_ref, g_ref, w_ref, b_ref, o_ref):
    deg = jnp.maximum(g_ref[0] + g_ref[1], 1.0)
    h = s_ref[0] + s_ref[1] + deg * x_ref[...]
    y = jnp.dot(h, w_ref[...], preferred_element_type=jnp.float32)
    o_ref[...] = jnp.maximum(y + b_ref[...], 0.0)


def _tc_final(x, s_part, deg3, W, b):
    BR = 400
    grid = (N_NODES // BR,)
    return pl.pallas_call(
        _tc_body,
        grid=grid,
        in_specs=[
            pl.BlockSpec((BR, D), lambda i: (i, 0)),
            pl.BlockSpec((NC, BR, D), lambda i: (0, i, 0)),
            pl.BlockSpec((NC, BR, 1), lambda i: (0, i, 0)),
            pl.BlockSpec((D, D), lambda i: (0, 0)),
            pl.BlockSpec((1, D), lambda i: (0, 0)),
        ],
        out_specs=pl.BlockSpec((BR, D), lambda i: (i, 0)),
        out_shape=jax.ShapeDtypeStruct((N_NODES, D), jnp.float32),
    )(x, s_part, deg3, W, b)


@jax.jit
def kernel(x, edge_d, edge_index, W, b):
    s_part, deg_part = _sc_scatter(x, edge_d, edge_index)
    deg3 = deg_part.reshape(NC, DEG_PAD, 1)
    return _tc_final(x, s_part, deg3, W, b.reshape(1, D))


# speedup vs baseline: 16.6422x; 1.0427x over previous
import jax
import jax.numpy as jnp
from jax import lax
from jax.experimental import pallas as pl
from jax.experimental.pallas import tpu as pltpu
from jax.experimental.pallas import tpu_sc as plsc

N_NODES = 10000
N_EDGES = 320000
D = 128

NC = 2
NS = 16
NW = NC * NS
EPW = N_EDGES // NW
EB = 80
NBATCH = EPW // EB
CHUNK = 25
NCHUNK = NBATCH // CHUNK
CE = CHUNK * EB
NBUF = 3
PF = 2
ROWS_PAD = 10240
RPT = ROWS_PAD // NS
DEG_PAD = ROWS_PAD
RPT_D = RPT


def _sc_body(ei, ed, x_hbm, s_out, deg_out,
             acc, degs, r0, r1, r2, srcs, dsts, dsv, ones_v,
             g0, g1, g2, s0, s1, s2, dsem):
    c = lax.axis_index("c")
    s = lax.axis_index("s")
    wid = c * NS + s
    rows = [r0, r1, r2]
    gsem = [g0, g1, g2]
    ssem = [s0, s1, s2]

    def zloop(i, _):
        zero16 = jnp.zeros((16,), jnp.float32)
        for cc in range(8):
            r0[i, pl.ds(cc * 16, 16)] = zero16
        return 0
    lax.fori_loop(0, EB, zloop, 0)
    for k in range(RPT // EB):
        pltpu.sync_copy(r0, acc.at[pl.ds(s * RPT + k * EB, EB)])
    for k in range(RPT_D // D):
        pltpu.sync_copy(r0.at[0], degs.at[pl.ds(s * RPT_D + k * D, D)])

    def oloop(i, _):
        ones_v[pl.ds(i * 16, 16)] = jnp.ones((16,), jnp.float32)
        return 0
    lax.fori_loop(0, EB // 16, oloop, 0)

    plsc.subcore_barrier()

    def start_gather(t, b):
        pltpu.async_copy(x_hbm.at[srcs.at[t]], rows[b], gsem[b])

    def wait_gather(b):
        pltpu.make_async_copy(x_hbm.at[srcs.at[0]], rows[b],
                              gsem[b]).wait()

    def start_scatter(t, b):
        pltpu.async_copy(rows[b], acc.at[dsts.at[t]], ssem[b], add=True)
        pltpu.async_copy(ones_v, degs.at[dsts.at[t]], dsem, add=True)

    def wait_scatter(b):
        pltpu.make_async_copy(rows[b], acc.at[dsts.at[0]],
                              ssem[b]).wait()

    def scale(t, b):
        r = rows[b]

        def body(g, _):
            svec = 1.0 - dsv[t, pl.ds(g * 16, 16)]
            for j in range(16):
                sc = svec[j]
                row = g * 16 + j
                for cc in range(8):
                    r[row, pl.ds(cc * 16, 16)] = (
                        r[row, pl.ds(cc * 16, 16)] * sc)
            return 0
        lax.fori_loop(0, EB // 16, body, 0)

    def step(t, b, ws, nxt):
        wait_gather(b)
        scale(t, b)
        start_scatter(t, b)
        b2 = (b + PF) % NBUF
        if ws:
            wait_scatter(b2)
        if nxt is not None:
            start_gather(nxt, b2)

    def chunk_body(k, _):
        pltpu.sync_copy(ei.at[0].at[wid].at[k], srcs)
        pltpu.sync_copy(ei.at[1].at[wid].at[k], dsts)
        pltpu.sync_copy(ed.at[wid].at[k], dsv)

        for t in range(PF):
            start_gather(t, t)
        step(0, 0, False, 2)
        step(1, 1, True, 3)
        step(2, 2, True, 4)

        def pipe(t3, _):
            ta = 3 * t3
            step(ta, 0, True, ta + PF)
            step(ta + 1, 1, True, ta + 1 + PF)
            step(ta + 2, 2, True, ta + 2 + PF)
            return 0
        lax.fori_loop(1, (CHUNK - 4) // NBUF, pipe, 0)

        step(21, 0, True, 23)
        step(22, 1, True, 24)
        step(23, 2, False, None)
        step(24, 0, False, None)
        wait_scatter(1)
        wait_scatter(2)
        wait_scatter(0)

        def ddrain(i, _):
            pltpu.make_async_copy(ones_v, degs.at[dsts.at[0]],
                                  dsem).wait()
            return 0
        lax.fori_loop(0, CHUNK, ddrain, 0)
        return 0

    lax.fori_loop(0, NCHUNK, chunk_body, 0)

    plsc.subcore_barrier()

    pltpu.sync_copy(acc.at[pl.ds(s * RPT, RPT)],
                    s_out.at[c].at[pl.ds(s * RPT, RPT)])
    pltpu.sync_copy(degs.at[pl.ds(s * RPT_D, RPT_D)],
                    deg_out.at[c].at[pl.ds(s * RPT_D, RPT_D)])


def _sc_scatter(x, edge_d, edge_index):
    mesh = plsc.VectorSubcoreMesh(core_axis_name="c", subcore_axis_name="s")
    f = pl.kernel(
        _sc_body,
        out_type=(
            jax.ShapeDtypeStruct((NC, ROWS_PAD, D), jnp.float32),
            jax.ShapeDtypeStruct((NC, DEG_PAD), jnp.float32),
        ),
        mesh=mesh,
        scratch_types=[
            pltpu.VMEM_SHARED((ROWS_PAD, D), jnp.float32),
            pltpu.VMEM_SHARED((DEG_PAD,), jnp.float32),
            pltpu.VMEM((EB, D), jnp.float32),
            pltpu.VMEM((EB, D), jnp.float32),
            pltpu.VMEM((EB, D), jnp.float32),
            pltpu.VMEM((CHUNK, EB), jnp.int32),
            pltpu.VMEM((CHUNK, EB), jnp.int32),
            pltpu.VMEM((CHUNK, EB), jnp.float32),
            pltpu.VMEM((EB,), jnp.float32),
            pltpu.SemaphoreType.DMA,
            pltpu.SemaphoreType.DMA,
            pltpu.SemaphoreType.DMA,
            pltpu.SemaphoreType.DMA,
            pltpu.SemaphoreType.DMA,
            pltpu.SemaphoreType.DMA,
            pltpu.SemaphoreType.DMA,
        ],
    )
    return f(edge_index.reshape(2, NW, NCHUNK, CHUNK, EB),
             edge_d.reshape(NW, NCHUNK, CHUNK, EB), x)


def _tc_body(x_ref, s_ref, g_ref, w_ref, b_ref, o_ref):
    deg = jnp.maximum(g_ref[0] + g_ref[1], 1.0)
    h = s_ref[0] + s_ref[1] + deg * x_ref[...]
    y = jnp.dot(h, w_ref[...], preferred_element_type=jnp.float32)
    o_ref[...] = jnp.maximum(y + b_ref[...], 0.0)


def _tc_final(x, s_part, deg3, W, b):
    BR = 1000
    grid = (N_NODES // BR,)
    return pl.pallas_call(
        _tc_body,
        grid=grid,
        in_specs=[
            pl.BlockSpec((BR, D), lambda i: (i, 0)),
            pl.BlockSpec((NC, BR, D), lambda i: (0, i, 0)),
            pl.BlockSpec((NC, BR, 1), lambda i: (0, i, 0)),
            pl.BlockSpec((D, D), lambda i: (0, 0)),
            pl.BlockSpec((1, D), lambda i: (0, 0)),
        ],
        out_specs=pl.BlockSpec((BR, D), lambda i: (i, 0)),
        out_shape=jax.ShapeDtypeStruct((N_NODES, D), jnp.float32),
    )(x, s_part, deg3, W, b)


@jax.jit
def kernel(x, edge_d, edge_index, W, b):
    s_part, deg_part = _sc_scatter(x, edge_d, edge_index)
    deg3 = deg_part.reshape(NC, DEG_PAD, 1)
    return _tc_final(x, s_part, deg3, W, b.reshape(1, D))


# speedup vs baseline: 17.3911x; 1.0450x over previous
import jax
import jax.numpy as jnp
from jax import lax
from jax.experimental import pallas as pl
from jax.experimental.pallas import tpu as pltpu
from jax.experimental.pallas import tpu_sc as plsc

N_NODES = 10000
N_EDGES = 320000
D = 128

NC = 2
NS = 16
NW = NC * NS
EPW = N_EDGES // NW
EB = 80
NBATCH = EPW // EB
CHUNK = 25
NCHUNK = NBATCH // CHUNK
CE = CHUNK * EB
NBUF = 3
PF = 2
ROWS_PAD = 10240
RPT = ROWS_PAD // NS
DEG_PAD = ROWS_PAD
RPT_D = RPT


def _sc_body(ei, ed, x_hbm, s_out, deg_out,
             acc, degs, r0, r1, r2, srcs, dsts, dsv, ones_v,
             g0, g1, g2, s0, s1, s2, dsem):
    c = lax.axis_index("c")
    s = lax.axis_index("s")
    wid = c * NS + s
    rows = [r0, r1, r2]
    gsem = [g0, g1, g2]
    ssem = [s0, s1, s2]

    def zloop(i, _):
        zero16 = jnp.zeros((16,), jnp.float32)
        for cc in range(8):
            r0[i, pl.ds(cc * 16, 16)] = zero16
        return 0
    lax.fori_loop(0, EB, zloop, 0)
    for k in range(RPT // EB):
        pltpu.sync_copy(r0, acc.at[pl.ds(s * RPT + k * EB, EB)])
    for k in range(RPT_D // D):
        pltpu.sync_copy(r0.at[0], degs.at[pl.ds(s * RPT_D + k * D, D)])

    def oloop(i, _):
        ones_v[pl.ds(i * 16, 16)] = jnp.ones((16,), jnp.float32)
        return 0
    lax.fori_loop(0, EB // 16, oloop, 0)

    plsc.subcore_barrier()

    def start_gather(t, b):
        pltpu.async_copy(x_hbm.at[srcs.at[t]], rows[b], gsem[b])

    def wait_gather(b):
        pltpu.make_async_copy(x_hbm.at[srcs.at[0]], rows[b],
                              gsem[b]).wait()

    def start_scatter(t, b):
        pltpu.async_copy(rows[b], acc.at[dsts.at[t]], ssem[b], add=True)
        pltpu.async_copy(ones_v, degs.at[dsts.at[t]], dsem, add=True)

    def wait_scatter(b):
        pltpu.make_async_copy(rows[b], acc.at[dsts.at[0]],
                              ssem[b]).wait()

    def scale(t, b):
        r = rows[b]

        def body(g, _):
            svec = 1.0 - dsv[t, pl.ds(g * 16, 16)]
            for j in range(16):
                sc = svec[j]
                row = g * 16 + j
                for cc in range(8):
                    r[row, pl.ds(cc * 16, 16)] = (
                        r[row, pl.ds(cc * 16, 16)] * sc)
            return 0
        lax.fori_loop(0, EB // 16, body, 0)

    def step(t, b, ws, nxt):
        wait_gather(b)
        scale(t, b)
        start_scatter(t, b)
        b2 = (b + PF) % NBUF
        if ws:
            wait_scatter(b2)
        if nxt is not None:
            start_gather(nxt, b2)

    def chunk_body(k, _):
        pltpu.sync_copy(ei.at[0].at[wid].at[k], srcs)
        pltpu.sync_copy(ei.at[1].at[wid].at[k], dsts)
        pltpu.sync_copy(ed.at[wid].at[k], dsv)

        for t in range(PF):
            start_gather(t, t)
        step(0, 0, False, 2)
        step(1, 1, True, 3)
        step(2, 2, True, 4)

        def pipe(t3, _):
            ta = 3 * t3
            step(ta, 0, True, ta + PF)
            step(ta + 1, 1, True, ta + 1 + PF)
            step(ta + 2, 2, True, ta + 2 + PF)
            return 0
        lax.fori_loop(1, (CHUNK - 4) // NBUF, pipe, 0)

        step(21, 0, True, 23)
        step(22, 1, True, 24)
        step(23, 2, False, None)
        step(24, 0, False, None)
        wait_scatter(1)
        wait_scatter(2)
        wait_scatter(0)

        def ddrain(i, _):
            pltpu.make_async_copy(ones_v, degs.at[dsts.at[0]],
                                  dsem).wait()
            return 0
        lax.fori_loop(0, CHUNK, ddrain, 0)
        return 0

    lax.fori_loop(0, NCHUNK, chunk_body, 0)

    plsc.subcore_barrier()

    pltpu.sync_copy(acc.at[pl.ds(s * RPT, RPT)],
                    s_out.at[c].at[pl.ds(s * RPT, RPT)])
    pltpu.sync_copy(degs.at[pl.ds(s * RPT_D, RPT_D)],
                    deg_out.at[c].at[pl.ds(s * RPT_D, RPT_D)])


def _sc_scatter(x, edge_d, edge_index):
    mesh = plsc.VectorSubcoreMesh(core_axis_name="c", subcore_axis_name="s")
    f = pl.kernel(
        _sc_body,
        out_type=(
            jax.ShapeDtypeStruct((NC, ROWS_PAD, D), jnp.float32),
            jax.ShapeDtypeStruct((NC, DEG_PAD), jnp.float32),
        ),
        mesh=mesh,
        scratch_types=[
            pltpu.VMEM_SHARED((ROWS_PAD, D), jnp.float32),
            pltpu.VMEM_SHARED((DEG_PAD,), jnp.float32),
            pltpu.VMEM((EB, D), jnp.float32),
            pltpu.VMEM((EB, D), jnp.float32),
            pltpu.VMEM((EB, D), jnp.float32),
            pltpu.VMEM((CHUNK, EB), jnp.int32),
            pltpu.VMEM((CHUNK, EB), jnp.int32),
            pltpu.VMEM((CHUNK, EB), jnp.float32),
            pltpu.VMEM((EB,), jnp.float32),
            pltpu.SemaphoreType.DMA,
            pltpu.SemaphoreType.DMA,
            pltpu.SemaphoreType.DMA,
            pltpu.SemaphoreType.DMA,
            pltpu.SemaphoreType.DMA,
            pltpu.SemaphoreType.DMA,
            pltpu.SemaphoreType.DMA,
        ],
    )
    return f(edge_index.reshape(2, NW, NCHUNK, CHUNK, EB),
             edge_d.reshape(NW, NCHUNK, CHUNK, EB), x)


def _tc_body(x_ref, s_ref, g_ref, w_ref, b_ref, o_ref):
    i = pl.program_id(0)
    g2 = jnp.transpose(g_ref[:, pl.ds(pl.multiple_of(i * 1280, 128), 1280)])
    deg = jnp.maximum(g2[:, 0:1] + g2[:, 1:2], 1.0)
    h = s_ref[0] + s_ref[1] + deg * x_ref[...]
    y = jnp.dot(h, w_ref[...], preferred_element_type=jnp.float32)
    o_ref[...] = jnp.maximum(y + b_ref[...], 0.0)


def _tc_final(x, s_part, deg_part, W, b):
    BR = 1280
    grid = ((N_NODES + BR - 1) // BR,)
    return pl.pallas_call(
        _tc_body,
        grid=grid,
        in_specs=[
            pl.BlockSpec((BR, D), lambda i: (i, 0)),
            pl.BlockSpec((NC, BR, D), lambda i: (0, i, 0)),
            pl.BlockSpec((NC, DEG_PAD), lambda i: (0, 0)),
            pl.BlockSpec((D, D), lambda i: (0, 0)),
            pl.BlockSpec((1, D), lambda i: (0, 0)),
        ],
        out_specs=pl.BlockSpec((BR, D), lambda i: (i, 0)),
        out_shape=jax.ShapeDtypeStruct((N_NODES, D), jnp.float32),
    )(x, s_part, deg_part, W, b)


@jax.jit
def kernel(x, edge_d, edge_index, W, b):
    s_part, deg_part = _sc_scatter(x, edge_d, edge_index)
    return _tc_final(x, s_part, deg_part, W, b.reshape(1, D))
